# baseline (speedup 1.0000x reference)
import functools
import math

import jax
import jax.numpy as jnp
import numpy as np
from jax.experimental import pallas as pl
from jax.experimental.pallas import tpu as pltpu

_NEG = -1e30
_CUM_BLK = 256


def _threefry2x32(k1, k2, x0, x1):
    def rotl(x, d):
        return ((x << np.uint32(d)) | (x >> np.uint32(32 - d))).astype(np.uint32)
    rotations = [(13, 15, 26, 6), (17, 29, 16, 24)]
    ks = [np.uint32(k1), np.uint32(k2),
          np.uint32(np.uint32(k1) ^ np.uint32(k2) ^ np.uint32(0x1BD11BDA))]
    x = [(x0 + ks[0]).astype(np.uint32), (x1 + ks[1]).astype(np.uint32)]
    for i in range(5):
        for r in rotations[i % 2]:
            x[0] = (x[0] + x[1]).astype(np.uint32)
            x[1] = rotl(x[1], r) ^ x[0]
        x[0] = (x[0] + ks[(i + 1) % 3]).astype(np.uint32)
        x[1] = (x[1] + ks[(i + 2) % 3] + np.uint32(i + 1)).astype(np.uint32)
    return x


def _np_randint(seed: int, shape, span: int):
    b1, b2 = _threefry2x32(np.uint32(0), np.uint32(seed),
                           np.zeros(2, np.uint32), np.arange(2, dtype=np.uint32))
    size = int(np.prod(shape))
    hi = np.zeros(size, np.uint32)
    lo = np.arange(size, dtype=np.uint32)
    def bits(k1, k2):
        a, b = _threefry2x32(k1, k2, hi, lo)
        return a ^ b
    higher, lower = bits(b1[0], b2[0]), bits(b1[1], b2[1])
    uspan = np.uint32(span)
    mult = np.uint32(np.uint64(2) ** 16 % uspan)
    mult = np.uint32((np.uint64(mult) * np.uint64(mult)) % np.uint64(uspan))
    off = ((higher % uspan) * mult + (lower % uspan)) % uspan
    return off.astype(np.int32).reshape(shape)


@functools.lru_cache(maxsize=None)
def _sample_counts_t(l_q: int, sample_k: int, l_k: int):
    idx = _np_randint(42, (l_q, sample_k), l_k)
    cnt = np.zeros((l_q, l_k), np.float32)
    np.add.at(cnt, (np.arange(l_q)[:, None], idx), 1.0)
    return np.ascontiguousarray(cnt.T)


@functools.lru_cache(maxsize=None)
def _sample_bias_t(l_q: int, sample_k: int, l_k: int):
    cnt_t = _sample_counts_t(l_q, sample_k, l_k)
    return np.where(cnt_t > 0, np.float32(0), np.float32(_NEG))


def _fused_body(n_top, l_q, l_k, d, hp, gh,
                q_ref, k_ref, v_ref, cnt_ref, bias_ref, out_ref,
                m_scr, idx_scr):
    g = pl.program_id(1)
    h_tot = m_scr.shape[0]

    @pl.when(g < gh)
    def _measure():
        s_ts = []
        for j in range(hp):
            q = q_ref[0, :, j * d:(j + 1) * d]
            k = k_ref[0, :, j * d:(j + 1) * d]
            s_ts.append(jax.lax.dot_general(k, q, (((1,), (1,)), ((), ())),
                                            preferred_element_type=jnp.float32))
        cnt_t = cnt_ref[...]
        bias_t = bias_ref[...]
        for j in range(hp):
            s_t = s_ts[j]
            smax = jnp.max(s_t + bias_t, axis=0, keepdims=True)
            ssum = jnp.sum(s_t * cnt_t, axis=0, keepdims=True)
            m_scr[pl.ds(g * hp + j, 1), :] = smax - ssum * (1.0 / l_k)

    @pl.when(g == gh - 1)
    def _topk():
        m = m_scr[...]
        lane = jax.lax.broadcasted_iota(jnp.int32, (h_tot, l_q), 1)
        ucol = jax.lax.broadcasted_iota(jnp.int32, (h_tot, n_top), 1)

        def body(i, carry):
            m_cur, idxm = carry
            rowmax = jnp.max(m_cur, axis=1, keepdims=True)
            best = jnp.min(jnp.where(m_cur == rowmax, lane, l_q), axis=1,
                           keepdims=True)
            idxm = jnp.where(ucol == i, best, idxm)
            m_cur = jnp.where(lane == best, _NEG, m_cur)
            return m_cur, idxm

        idx0 = jnp.zeros((h_tot, n_top), jnp.int32)
        _, idxm = jax.lax.fori_loop(0, n_top, body, (m, idx0))
        idx_scr[...] = idxm

    @pl.when(g >= gh)
    def _attend():
        cb = _CUM_BLK
        rr = jax.lax.broadcasted_iota(jnp.int32, (cb, cb), 0)
        cc = jax.lax.broadcasted_iota(jnp.int32, (cb, cb), 1)
        tril_b = (rr >= cc).astype(jnp.float32)
        for j in range(hp):
            q = q_ref[0, :, j * d:(j + 1) * d]
            k = k_ref[0, :, j * d:(j + 1) * d]
            v = v_ref[0, :, j * d:(j + 1) * d]
            idxrow = idx_scr[pl.ds((g - gh) * hp + j, 1), :]
            rows = jax.lax.broadcasted_iota(jnp.int32, (l_q, n_top), 0)
            p = (rows == idxrow).astype(jnp.float32)
            qsel = jax.lax.dot_general(p, q, (((0,), (0,)), ((), ())),
                                       preferred_element_type=jnp.float32)
            stop = jax.lax.dot_general(qsel, k, (((1,), (1,)), ((), ())),
                                       preferred_element_type=jnp.float32)
            rmax = jnp.max(stop, axis=1, keepdims=True)
            e = jnp.exp(stop - rmax)
            attn = e / jnp.sum(e, axis=1, keepdims=True)
            upd = jnp.dot(attn, v, preferred_element_type=jnp.float32)
            running = jnp.zeros((1, d), jnp.float32)
            ctx_blocks = []
            for bk in range(l_q // cb):
                vb = v[bk * cb:(bk + 1) * cb, :]
                fine = jnp.dot(tril_b, vb, preferred_element_type=jnp.float32)
                ctx_blocks.append(fine + running)
                running = running + fine[cb - 1:cb, :]
            ctx = jnp.concatenate(ctx_blocks, axis=0)
            placed = jnp.dot(p, upd, preferred_element_type=jnp.float32)
            selb = jnp.dot(p, jnp.ones((n_top, d), jnp.float32),
                           preferred_element_type=jnp.float32)
            out_ref[0, :, j * d:(j + 1) * d] = ctx * (1.0 - selb) + placed


def kernel(queries, keys, values):
    b, l_q, h, d = queries.shape
    l_k = keys.shape[1]
    u = 5 * int(math.ceil(math.log(l_k)))
    hp = max(1, 128 // d)
    assert h % hp == 0
    gh = h // hp
    cnt_t = jnp.asarray(_sample_counts_t(l_q, u, l_k))
    bias_t = jnp.asarray(_sample_bias_t(l_q, u, l_k))

    q2 = queries.reshape(b, l_q, h * d)
    k2 = keys.reshape(b, l_k, h * d)
    v2 = values.reshape(b, l_k, h * d)

    slab = pl.BlockSpec((1, l_q, hp * d), lambda bi, g: (bi, 0, g % gh))
    att = pl.BlockSpec((1, l_q, hp * d),
                       lambda bi, g: (bi, 0, jnp.maximum(g - gh, 0)))
    const = pl.BlockSpec((l_k, l_q), lambda bi, g: (0, 0))

    out = pl.pallas_call(
        functools.partial(_fused_body, u, l_q, l_k, d, hp, gh),
        grid=(b, 2 * gh),
        in_specs=[slab, slab, att, const, const],
        out_specs=att,
        out_shape=jax.ShapeDtypeStruct((b, l_q, h * d), jnp.float32),
        scratch_shapes=[pltpu.VMEM((h, l_q), jnp.float32),
                        pltpu.VMEM((h, u), jnp.int32)],
        compiler_params=pltpu.CompilerParams(
            dimension_semantics=("arbitrary", "arbitrary")),
    )(q2, k2, v2, cnt_t, bias_t)
    return out.reshape(b, l_q, h, d)


# speedup vs baseline: 1.0162x; 1.0162x over previous
import functools
import math

import jax
import jax.numpy as jnp
import numpy as np
from jax.experimental import pallas as pl
from jax.experimental.pallas import tpu as pltpu

_NEG = -1e30
_CUM_BLK = 256


def _threefry2x32(k1, k2, x0, x1):
    def rotl(x, d):
        return ((x << np.uint32(d)) | (x >> np.uint32(32 - d))).astype(np.uint32)
    rotations = [(13, 15, 26, 6), (17, 29, 16, 24)]
    ks = [np.uint32(k1), np.uint32(k2),
          np.uint32(np.uint32(k1) ^ np.uint32(k2) ^ np.uint32(0x1BD11BDA))]
    x = [(x0 + ks[0]).astype(np.uint32), (x1 + ks[1]).astype(np.uint32)]
    for i in range(5):
        for r in rotations[i % 2]:
            x[0] = (x[0] + x[1]).astype(np.uint32)
            x[1] = rotl(x[1], r) ^ x[0]
        x[0] = (x[0] + ks[(i + 1) % 3]).astype(np.uint32)
        x[1] = (x[1] + ks[(i + 2) % 3] + np.uint32(i + 1)).astype(np.uint32)
    return x


def _np_randint(seed: int, shape, span: int):
    b1, b2 = _threefry2x32(np.uint32(0), np.uint32(seed),
                           np.zeros(2, np.uint32), np.arange(2, dtype=np.uint32))
    size = int(np.prod(shape))
    hi = np.zeros(size, np.uint32)
    lo = np.arange(size, dtype=np.uint32)
    def bits(k1, k2):
        a, b = _threefry2x32(k1, k2, hi, lo)
        return a ^ b
    higher, lower = bits(b1[0], b2[0]), bits(b1[1], b2[1])
    uspan = np.uint32(span)
    mult = np.uint32(np.uint64(2) ** 16 % uspan)
    mult = np.uint32((np.uint64(mult) * np.uint64(mult)) % np.uint64(uspan))
    off = ((higher % uspan) * mult + (lower % uspan)) % uspan
    return off.astype(np.int32).reshape(shape)


@functools.lru_cache(maxsize=None)
def _sample_counts_t(l_q: int, sample_k: int, l_k: int):
    idx = _np_randint(42, (l_q, sample_k), l_k)
    cnt = np.zeros((l_q, l_k), np.float32)
    np.add.at(cnt, (np.arange(l_q)[:, None], idx), 1.0)
    return np.ascontiguousarray(cnt.T)


@functools.lru_cache(maxsize=None)
def _sample_bias_t(l_q: int, sample_k: int, l_k: int):
    cnt_t = _sample_counts_t(l_q, sample_k, l_k)
    return np.where(cnt_t > 0, np.float32(0), np.float32(_NEG))


def _fused_body(n_top, l_q, l_k, d, hp, gh,
                q_ref, k_ref, v_ref, cnt_ref, bias_ref, out_ref,
                m_scr, idx_scr):
    g = pl.program_id(1)
    h_tot = m_scr.shape[0]

    @pl.when(g < gh)
    def _measure():
        s_ts = []
        for j in range(hp):
            q = q_ref[0, :, j * d:(j + 1) * d]
            k = k_ref[0, :, j * d:(j + 1) * d]
            s_ts.append(jax.lax.dot_general(k, q, (((1,), (1,)), ((), ())),
                                            preferred_element_type=jnp.float32))
        cnt_t = cnt_ref[...]
        bias_t = bias_ref[...]
        for j in range(hp):
            s_t = s_ts[j]
            smax = jnp.max(s_t + bias_t, axis=0, keepdims=True)
            ssum = jnp.sum(s_t * cnt_t, axis=0, keepdims=True)
            m_scr[pl.ds(g * hp + j, 1), :] = smax - ssum * (1.0 / l_k)

    @pl.when(g == gh - 1)
    def _topk():
        m = m_scr[...]
        lane = jax.lax.broadcasted_iota(jnp.int32, (h_tot, l_q), 1)
        ucol = jax.lax.broadcasted_iota(jnp.int32, (h_tot, n_top), 1)

        def body(i, carry):
            m_cur, idxm = carry
            rowmax = jnp.max(m_cur, axis=1, keepdims=True)
            best = jnp.min(jnp.where(m_cur == rowmax, lane, l_q), axis=1,
                           keepdims=True)
            idxm = jnp.where(ucol == i, best, idxm)
            m_cur = jnp.where(lane == best, _NEG, m_cur)
            return m_cur, idxm

        idx0 = jnp.zeros((h_tot, n_top), jnp.int32)
        _, idxm = jax.lax.fori_loop(0, n_top, body, (m, idx0))
        idx_scr[...] = idxm

    @pl.when(g >= gh)
    def _attend():
        cb = _CUM_BLK
        rr = jax.lax.broadcasted_iota(jnp.int32, (cb, cb), 0)
        cc = jax.lax.broadcasted_iota(jnp.int32, (cb, cb), 1)
        tril_b = (rr >= cc).astype(jnp.float32)
        for j in range(hp):
            q = q_ref[0, :, j * d:(j + 1) * d]
            k = k_ref[0, :, j * d:(j + 1) * d]
            v = v_ref[0, :, j * d:(j + 1) * d]
            idxrow = idx_scr[pl.ds((g - gh) * hp + j, 1), :]
            rows = jax.lax.broadcasted_iota(jnp.int32, (l_q, n_top), 0)
            p = (rows == idxrow).astype(jnp.float32)
            qsel = jax.lax.dot_general(p, q, (((0,), (0,)), ((), ())),
                                       preferred_element_type=jnp.float32)
            stop = jax.lax.dot_general(qsel, k, (((1,), (1,)), ((), ())),
                                       preferred_element_type=jnp.float32)
            rmax = jnp.max(stop, axis=1, keepdims=True)
            e = jnp.exp(stop - rmax)
            attn = e / jnp.sum(e, axis=1, keepdims=True)
            upd = jnp.dot(attn, v, preferred_element_type=jnp.float32)
            running = jnp.zeros((1, d), jnp.float32)
            ctx_blocks = []
            for bk in range(l_q // cb):
                vb = v[bk * cb:(bk + 1) * cb, :]
                fine = jnp.dot(tril_b, vb, preferred_element_type=jnp.float32)
                ctx_blocks.append(fine + running)
                running = running + fine[cb - 1:cb, :]
            ctx = jnp.concatenate(ctx_blocks, axis=0)
            placed = jnp.dot(p, upd, preferred_element_type=jnp.float32)
            anysel = jnp.max(p, axis=1, keepdims=True)
            out_ref[0, :, j * d:(j + 1) * d] = ctx * (1.0 - anysel) + placed


def kernel(queries, keys, values):
    b, l_q, h, d = queries.shape
    l_k = keys.shape[1]
    u = 5 * int(math.ceil(math.log(l_k)))
    hp = max(1, 128 // d)
    assert h % hp == 0
    gh = h // hp
    cnt_t = jnp.asarray(_sample_counts_t(l_q, u, l_k))
    bias_t = jnp.asarray(_sample_bias_t(l_q, u, l_k))

    q2 = queries.reshape(b, l_q, h * d)
    k2 = keys.reshape(b, l_k, h * d)
    v2 = values.reshape(b, l_k, h * d)

    slab = pl.BlockSpec((1, l_q, hp * d), lambda bi, g: (bi, 0, g % gh))
    att = pl.BlockSpec((1, l_q, hp * d),
                       lambda bi, g: (bi, 0, jnp.maximum(g - gh, 0)))
    const = pl.BlockSpec((l_k, l_q), lambda bi, g: (0, 0))

    out = pl.pallas_call(
        functools.partial(_fused_body, u, l_q, l_k, d, hp, gh),
        grid=(b, 2 * gh),
        in_specs=[slab, slab, att, const, const],
        out_specs=att,
        out_shape=jax.ShapeDtypeStruct((b, l_q, h * d), jnp.float32),
        scratch_shapes=[pltpu.VMEM((h, l_q), jnp.float32),
                        pltpu.VMEM((h, u), jnp.int32)],
        compiler_params=pltpu.CompilerParams(
            dimension_semantics=("arbitrary", "arbitrary")),
    )(q2, k2, v2, cnt_t, bias_t)
    return out.reshape(b, l_q, h, d)
